# baseline (speedup 1.0000x reference)
import functools
import math

import jax
import jax.numpy as jnp
from jax import lax
from jax.experimental import pallas as pl
from jax.experimental.pallas import tpu as pltpu
from jax.experimental.pallas import tpu_sc as plsc

VOCAB = 1000000
EMB = 64
B = 4096
L = 200
SCALE = math.sqrt(EMB)

_info = plsc.get_sparse_core_info()
NC, NS, LANES = _info.num_cores, _info.num_subcores, _info.num_lanes
NW = NC * NS
BW = B // NW
EHI = EMB // 8
BBLK = B // BW
NB = 4
GROUPS = L // NB


def _body(tok_hbm, table_hbm, out_hbm, idx_raw, idx_t, gbuf, tbuf, gsems, wsems):
    wid = lax.axis_index("s") * NC + lax.axis_index("c")
    iota = lax.iota(jnp.int32, LANES)

    def gather_start(l, b):
        for t0 in range(0, BW, LANES):
            idxv = idx_t[l, pl.ds(t0, LANES)]
            pltpu.async_copy(
                table_hbm.at[idxv], gbuf.at[b, pl.ds(t0, LANES)], gsems[b]
            )

    def gather_wait(l, b):
        pltpu.make_async_copy(table_hbm.at[idx_t.at[l]], gbuf.at[b], gsems[b]).wait()

    def wb_start(l, b):
        for eh in range(EHI):
            pltpu.async_copy(
                tbuf.at[b, pl.ds(eh * 8, 8)], out_hbm.at[l, eh, wid], wsems[b]
            )

    def wb_wait(l, b):
        for eh in range(EHI):
            pltpu.make_async_copy(
                tbuf.at[b, pl.ds(eh * 8, 8)], out_hbm.at[l, eh, wid], wsems[b]
            ).wait()

    diag = [(iota + k) & 15 for k in range(LANES)]

    def tscale(b):
        def body_eb(ebi, c):
            e_b = ebi * LANES
            for t0 in range(0, BW, LANES):
                tv = iota + t0
                for k in range(LANES):
                    ev = diag[k] + e_b
                    v = plsc.load_gather(gbuf.at[b], [tv, ev])
                    plsc.store_scatter(tbuf.at[b], [ev, tv], v * SCALE)
            return c

        lax.fori_loop(0, EMB // LANES, body_eb, 0)

    pltpu.sync_copy(tok_hbm.at[pl.ds(wid * BW, BW)], idx_raw)

    def tr_l(l, c):
        for t0 in range(0, BW, LANES):
            v = plsc.load_gather(
                idx_raw, [iota + t0, jnp.full((LANES,), l, jnp.int32)]
            )
            idx_t[l, pl.ds(t0, LANES)] = v
        return c

    lax.fori_loop(0, L, tr_l, 0)

    for b in range(NB):
        gather_start(b, b)

    for b in range(NB):
        gather_wait(b, b)
        tscale(b)
        wb_start(b, b)

    def group(go, carry):
        for b in range(NB):
            l = go * NB + b
            wb_wait(l - NB, b)
            gather_start(l, b)
        for b in range(NB):
            l = go * NB + b
            gather_wait(l, b)
            tscale(b)
            wb_start(l, b)
        return carry

    lax.fori_loop(1, GROUPS, group, 0)

    for b in range(NB):
        wb_wait((GROUPS - 1) * NB + b, b)


@functools.partial(jax.jit, static_argnames=())
def kernel(tokens, table):
    mesh = plsc.VectorSubcoreMesh(core_axis_name="c", subcore_axis_name="s")
    run = pl.kernel(
        _body,
        out_type=jax.ShapeDtypeStruct((L, EHI, BBLK, 8, BW), jnp.float32),
        mesh=mesh,
        scratch_types=[
            pltpu.VMEM((BW, L), jnp.int32),
            pltpu.VMEM((L, BW), jnp.int32),
            pltpu.VMEM((NB, BW, EMB), jnp.float32),
            pltpu.VMEM((NB, EMB, BW), jnp.float32),
            [pltpu.SemaphoreType.DMA] * NB,
            [pltpu.SemaphoreType.DMA] * NB,
        ],
        compiler_params=pltpu.CompilerParams(
            use_tc_tiling_on_sc=False, needs_layout_passes=False
        ),
    )
    out5 = run(tokens.astype(jnp.int32), table)
    return out5.transpose(2, 4, 0, 1, 3).reshape(B, L, EMB)


# speedup vs baseline: 1.0204x; 1.0204x over previous
import functools
import math

import jax
import jax.numpy as jnp
from jax import lax
from jax.experimental import pallas as pl
from jax.experimental.pallas import tpu as pltpu
from jax.experimental.pallas import tpu_sc as plsc

VOCAB = 1000000
EMB = 64
B = 4096
L = 200
SCALE = math.sqrt(EMB)

_info = plsc.get_sparse_core_info()
NC, NS, LANES = _info.num_cores, _info.num_subcores, _info.num_lanes
NW = NC * NS
BW = B // NW
EHI = EMB // 8
BBLK = B // BW
NB = 4
GROUPS = L // NB


def _body(tok_hbm, table_hbm, out_hbm, idx_raw, idx_t, gbuf, tbuf, gsems, wsems):
    wid = lax.axis_index("s") * NC + lax.axis_index("c")
    iota = lax.iota(jnp.int32, LANES)

    def gather_start(l, b):
        for t0 in range(0, BW, LANES):
            idxv = idx_t[l, pl.ds(t0, LANES)]
            pltpu.async_copy(
                table_hbm.at[idxv], gbuf.at[b, pl.ds(t0, LANES)], gsems[b]
            )

    def gather_wait(l, b):
        pltpu.make_async_copy(table_hbm.at[idx_t.at[l]], gbuf.at[b], gsems[b]).wait()

    def wb_start(l, b):
        for eh in range(EHI):
            pltpu.async_copy(
                tbuf.at[b, pl.ds(eh * 8, 8), pl.ds(0, BW)], out_hbm.at[l, eh, wid], wsems[b]
            )

    def wb_wait(l, b):
        for eh in range(EHI):
            pltpu.make_async_copy(
                tbuf.at[b, pl.ds(eh * 8, 8), pl.ds(0, BW)], out_hbm.at[l, eh, wid], wsems[b]
            ).wait()

    def tscale(b):
        def body_t(t, c):
            tv = jnp.full((LANES,), t, jnp.int32)
            for e0 in range(0, EMB, LANES):
                v = gbuf[b, t, pl.ds(e0, LANES)]
                plsc.store_scatter(tbuf.at[b], [iota + e0, tv], v * SCALE)
            return c

        lax.fori_loop(0, BW, body_t, 0)

    pltpu.sync_copy(tok_hbm.at[pl.ds(wid * BW, BW)], idx_raw)

    def tr_l(l, c):
        for t0 in range(0, BW, LANES):
            v = plsc.load_gather(
                idx_raw, [iota + t0, jnp.full((LANES,), l, jnp.int32)]
            )
            idx_t[l, pl.ds(t0, LANES)] = v
        return c

    lax.fori_loop(0, L, tr_l, 0)

    for b in range(NB):
        gather_start(b, b)

    for b in range(NB):
        gather_wait(b, b)
        tscale(b)
        wb_start(b, b)

    def group(go, carry):
        for b in range(NB):
            l = go * NB + b
            wb_wait(l - NB, b)
            gather_start(l, b)
        for b in range(NB):
            l = go * NB + b
            gather_wait(l, b)
            tscale(b)
            wb_start(l, b)
        return carry

    lax.fori_loop(1, GROUPS, group, 0)

    for b in range(NB):
        wb_wait((GROUPS - 1) * NB + b, b)


@functools.partial(jax.jit, static_argnames=())
def kernel(tokens, table):
    mesh = plsc.VectorSubcoreMesh(core_axis_name="c", subcore_axis_name="s")
    run = pl.kernel(
        _body,
        out_type=jax.ShapeDtypeStruct((L, EHI, BBLK, 8, BW), jnp.float32),
        mesh=mesh,
        scratch_types=[
            pltpu.VMEM((BW, L), jnp.int32),
            pltpu.VMEM((L, BW), jnp.int32),
            pltpu.VMEM((NB, BW, EMB), jnp.float32),
            pltpu.VMEM((NB, EMB, BW + 1), jnp.float32),
            [pltpu.SemaphoreType.DMA] * NB,
            [pltpu.SemaphoreType.DMA] * NB,
        ],
        compiler_params=pltpu.CompilerParams(
            use_tc_tiling_on_sc=False, needs_layout_passes=False
        ),
    )
    out5 = run(tokens.astype(jnp.int32), table)
    return out5.transpose(2, 4, 0, 1, 3).reshape(B, L, EMB)
